# Initial kernel scaffold; baseline (speedup 1.0000x reference)
import jax
import jax.numpy as jnp
from jax.experimental import pallas as pl


def kernel(n_in_feats, r_feats, edge_src, edge_dst, etype, norm, out_edges_mask, in_edges_mask, W_O, b_O, W_I, b_I, W_S, b_S, W_R, b_R, loop_rel, bn_gamma, bn_beta):
    raise NotImplementedError("write your pallas kernel here")


# speedup vs baseline: 5.4992x; 5.4992x over previous
import jax
import jax.numpy as jnp
from jax import lax
from jax.experimental import pallas as pl
from jax.experimental.pallas import tpu as pltpu
from jax.experimental.pallas import tpu_sc as plsc

N = 10000
E = 320000
D_IN = 128
D_OUT = 128
R = 16
EPS_ = 1e-5
DH = 64
TWO_N = 2 * N
SW = 20
NTILES = 16
EPT = E // NTILES
B = 80
NB = EPT // B
RPT = TWO_N // NTILES
CB = 125
SPT = (TWO_N * SW) // NTILES
SCB = 1000


def _sc_body(ncat, src_a, dst_a, dir_a, et_a, nrm_a, t_out, s_out,
             srcv, dstv, dirv, etv, nrmv, onesb, gv, jv, siv, civ,
             rows_v, cbuf, sbuf, tacc, sacc, sem):
    cid = lax.axis_index("c")
    sid = lax.axis_index("s")
    zf = jnp.zeros((16,), jnp.float32)
    ones16 = jnp.ones((16,), jnp.float32)

    for k in range(B // 16):
        onesb[pl.ds(16 * k, 16)] = ones16

    def _zrow(r_, _):
        for l in range(DH // 16):
            cbuf[r_, pl.ds(l * 16, 16)] = zf
        return 0

    lax.fori_loop(0, CB, _zrow, 0)

    def _zsbuf(i, _):
        sbuf[pl.ds(i * 16, 16)] = zf
        return 0

    lax.fori_loop(0, SCB // 16, _zsbuf, 0)

    def _zcopy(i, _):
        base = sid * RPT + i * CB
        pltpu.sync_copy(cbuf, tacc.at[pl.ds(base, CB)])
        return 0

    lax.fori_loop(0, RPT // CB, _zcopy, 0)

    def _zscopy(i, _):
        pltpu.sync_copy(sbuf, sacc.at[pl.ds(sid * SPT + i * SCB, SCB)])
        return 0

    lax.fori_loop(0, SPT // SCB, _zscopy, 0)
    plsc.subcore_barrier()

    def _batch(b, _):
        o = sid * EPT + b * B
        c1 = pltpu.async_copy(src_a.at[pl.ds(o, B)], srcv, sem)
        c2 = pltpu.async_copy(dst_a.at[pl.ds(o, B)], dstv, sem)
        c3 = pltpu.async_copy(dir_a.at[pl.ds(o, B)], dirv, sem)
        c4 = pltpu.async_copy(et_a.at[pl.ds(o, B)], etv, sem)
        c5 = pltpu.async_copy(nrm_a.at[pl.ds(o, B)], nrmv, sem)
        c1.wait(); c2.wait(); c3.wait(); c4.wait(); c5.wait()
        for k in range(B // 16):
            ds16 = pl.ds(16 * k, 16)
            j16 = dstv[ds16] + (1 - dirv[ds16]) * N
            gv[ds16] = srcv[ds16] + cid * N
            jv[ds16] = j16
            siv[ds16] = j16 * SW + etv[ds16]
            civ[ds16] = j16 * SW + R
        pltpu.async_copy(ncat.at[gv], rows_v, sem).wait()
        pltpu.sync_copy(rows_v, tacc.at[jv], add=True)

        @pl.when(cid == 0)
        def _():
            pltpu.sync_copy(nrmv, sacc.at[siv], add=True)
            pltpu.sync_copy(onesb, sacc.at[civ], add=True)

        return 0

    lax.fori_loop(0, NB, _batch, 0)
    plsc.subcore_barrier()

    def _tcopy(i, _):
        base = sid * RPT + i * CB
        pltpu.sync_copy(tacc.at[pl.ds(base, CB)], cbuf)
        pltpu.sync_copy(cbuf, t_out.at[pl.ds(cid * TWO_N + base, CB)])
        return 0

    lax.fori_loop(0, RPT // CB, _tcopy, 0)

    @pl.when(cid == 0)
    def _():
        def _scopy(i, _):
            base = sid * SPT + i * SCB
            pltpu.sync_copy(sacc.at[pl.ds(base, SCB)], sbuf)
            pltpu.sync_copy(sbuf, s_out.at[pl.ds(base, SCB)])
            return 0

        lax.fori_loop(0, SPT // SCB, _scopy, 0)


_sc_aggregate = pl.kernel(
    _sc_body,
    out_type=(
        jax.ShapeDtypeStruct((2 * TWO_N, DH), jnp.float32),
        jax.ShapeDtypeStruct((TWO_N * SW,), jnp.float32),
    ),
    mesh=plsc.VectorSubcoreMesh(core_axis_name="c", subcore_axis_name="s"),
    compiler_params=pltpu.CompilerParams(use_tc_tiling_on_sc=False),
    scratch_types=[
        pltpu.VMEM((B,), jnp.int32),
        pltpu.VMEM((B,), jnp.int32),
        pltpu.VMEM((B,), jnp.int32),
        pltpu.VMEM((B,), jnp.int32),
        pltpu.VMEM((B,), jnp.float32),
        pltpu.VMEM((B,), jnp.float32),
        pltpu.VMEM((B,), jnp.int32),
        pltpu.VMEM((B,), jnp.int32),
        pltpu.VMEM((B,), jnp.int32),
        pltpu.VMEM((B,), jnp.int32),
        pltpu.VMEM((B, DH), jnp.float32),
        pltpu.VMEM((CB, DH), jnp.float32),
        pltpu.VMEM((SCB,), jnp.float32),
        pltpu.VMEM_SHARED((TWO_N, DH), jnp.float32),
        pltpu.VMEM_SHARED((TWO_N * SW,), jnp.float32),
        pltpu.SemaphoreType.DMA,
    ],
)


def _tc_body(nin, t, s, rf, loop, wo, bo, wi, bi, ws, bs, wr, br, g, bb,
             nout, rout):
    r = rf[...]
    ao = jnp.concatenate([t[0:N], t[TWO_N:TWO_N + N]], axis=1)
    ai = jnp.concatenate([t[N:TWO_N], t[TWO_N + N:2 * TWO_N]], axis=1)
    so = s[0:N, 0:R]
    si = s[N:TWO_N, 0:R]
    co = s[0:N, R:R + 1]
    ci = s[N:TWO_N, R:R + 1]
    mo = ao - jnp.dot(so, r, preferred_element_type=jnp.float32)
    mi = ai - jnp.dot(si, r, preferred_element_type=jnp.float32)
    comp = (jnp.dot(mo, wo[...], preferred_element_type=jnp.float32)
            + jnp.dot(mi, wi[...], preferred_element_type=jnp.float32)
            + co * bo[...] + ci * bi[...])
    h = jnp.dot(nin[...] - loop[...], ws[...],
                preferred_element_type=jnp.float32) + bs[...] + comp
    h = h * (1.0 / 3.0)
    mean = jnp.mean(h, axis=0, keepdims=True)
    var = jnp.mean((h - mean) ** 2, axis=0, keepdims=True)
    y = (h - mean) * lax.rsqrt(var + EPS_) * g[...] + bb[...]
    nout[...] = jnp.tanh(y)
    rout[...] = jnp.dot(r, wr[...], preferred_element_type=jnp.float32) + br[...]


_tc_finish = pl.pallas_call(
    _tc_body,
    out_shape=(
        jax.ShapeDtypeStruct((N, D_OUT), jnp.float32),
        jax.ShapeDtypeStruct((R, D_OUT), jnp.float32),
    ),
)


def kernel(n_in_feats, r_feats, edge_src, edge_dst, etype, norm,
           out_edges_mask, in_edges_mask,
           W_O, b_O, W_I, b_I, W_S, b_S, W_R, b_R,
           loop_rel, bn_gamma, bn_beta):
    src = edge_src.astype(jnp.int32)
    dst = edge_dst.astype(jnp.int32)
    et = etype.astype(jnp.int32)
    dirv = out_edges_mask.astype(jnp.int32)
    nrm = norm.reshape(E)
    ncat = jnp.concatenate([n_in_feats[:, :DH], n_in_feats[:, DH:]], axis=0)
    t, s = _sc_aggregate(ncat, src, dst, dirv, et, nrm)
    s2 = s.reshape(TWO_N, SW)
    n_out, r_out = _tc_finish(n_in_feats, t, s2, r_feats, loop_rel,
                              W_O, b_O, W_I, b_I, W_S, b_S, W_R, b_R,
                              bn_gamma, bn_beta)
    return n_out, r_out


# speedup vs baseline: 8.9457x; 1.6267x over previous
import jax
import jax.numpy as jnp
from jax import lax
from jax.experimental import pallas as pl
from jax.experimental.pallas import tpu as pltpu
from jax.experimental.pallas import tpu_sc as plsc

N = 10000
E = 320000
D_IN = 128
D_OUT = 128
R = 16
EPS_ = 1e-5
DH = 64
TWO_N = 2 * N
SW = 20
NTILES = 16
EPT = E // NTILES
B = 80
NB = EPT // B
RPT = TWO_N // NTILES
CB = 125
SPT = (TWO_N * SW) // NTILES
SCB = 1000


def _sc_body(ncat, src_a, dst_a, dir_a, et_a, nrm_a, t_out, s_out,
             srcv0, srcv1, dstv0, dstv1, dirv0, dirv1, etv0, etv1,
             nrmv0, nrmv1, gv0, gv1, jv0, jv1, siv0, siv1, civ0, civ1,
             rows0, rows1, onesb, cbuf, sbuf, tacc, sacc,
             sem_m, sem_g, sem_s0, sem_s1):
    cid = lax.axis_index("c")
    sid = lax.axis_index("s")
    zf = jnp.zeros((16,), jnp.float32)
    ones16 = jnp.ones((16,), jnp.float32)

    for k in range(B // 16):
        onesb[pl.ds(16 * k, 16)] = ones16

    def _zrow(r_, _):
        for l in range(DH // 16):
            cbuf[r_, pl.ds(l * 16, 16)] = zf
        return 0

    lax.fori_loop(0, CB, _zrow, 0)

    def _zsbuf(i, _):
        sbuf[pl.ds(i * 16, 16)] = zf
        return 0

    lax.fori_loop(0, SCB // 16, _zsbuf, 0)

    def _zcopy(i, _):
        base = sid * RPT + i * CB
        pltpu.sync_copy(cbuf, tacc.at[pl.ds(base, CB)])
        return 0

    lax.fori_loop(0, RPT // CB, _zcopy, 0)

    def _zscopy(i, _):
        pltpu.sync_copy(sbuf, sacc.at[pl.ds(sid * SPT + i * SCB, SCB)])
        return 0

    lax.fori_loop(0, SPT // SCB, _zscopy, 0)
    plsc.subcore_barrier()

    bufsets = (
        (srcv0, dstv0, dirv0, etv0, nrmv0, gv0, jv0, siv0, civ0, rows0, sem_s0),
        (srcv1, dstv1, dirv1, etv1, nrmv1, gv1, jv1, siv1, civ1, rows1, sem_s1),
    )

    def _fire_meta(bn, q):
        sv, dv, rv, ev, nv = bufsets[q][0:5]
        o = sid * EPT + bn * B
        pltpu.async_copy(src_a.at[pl.ds(o, B)], sv, sem_m)
        pltpu.async_copy(dst_a.at[pl.ds(o, B)], dv, sem_m)
        pltpu.async_copy(dir_a.at[pl.ds(o, B)], rv, sem_m)
        pltpu.async_copy(et_a.at[pl.ds(o, B)], ev, sem_m)
        pltpu.async_copy(nrm_a.at[pl.ds(o, B)], nv, sem_m)

    def _wait_meta(q):
        sv, dv, rv, ev, nv = bufsets[q][0:5]
        pltpu.make_async_copy(src_a.at[pl.ds(0, B)], sv, sem_m).wait()
        pltpu.make_async_copy(dst_a.at[pl.ds(0, B)], dv, sem_m).wait()
        pltpu.make_async_copy(dir_a.at[pl.ds(0, B)], rv, sem_m).wait()
        pltpu.make_async_copy(et_a.at[pl.ds(0, B)], ev, sem_m).wait()
        pltpu.make_async_copy(nrm_a.at[pl.ds(0, B)], nv, sem_m).wait()

    def _wait_scatter(q):
        _, _, _, _, nv, _, jvq, sivq, civq, rowsq, sem_s = bufsets[q]
        pltpu.make_async_copy(rowsq, tacc.at[jvq], sem_s).wait()

        @pl.when(cid == 0)
        def _():
            pltpu.make_async_copy(nv, sacc.at[sivq], sem_s).wait()
            pltpu.make_async_copy(onesb, sacc.at[civq], sem_s).wait()

    _fire_meta(0, 0)

    def _group(g, _):
        for q in (0, 1):
            b = g * 2 + q
            sv, dv, rv, ev, nv, gvq, jvq, sivq, civq, rowsq, sem_s = bufsets[q]
            _wait_meta(q)

            @pl.when(g >= 1)
            def _():
                _wait_scatter(q)

            for k in range(B // 16):
                ds16 = pl.ds(16 * k, 16)
                j16 = dv[ds16] + (1 - rv[ds16]) * N
                gvq[ds16] = sv[ds16] + cid * N
                jvq[ds16] = j16
                sivq[ds16] = j16 * SW + ev[ds16]
                civq[ds16] = j16 * SW + R

            @pl.when(b < NB - 1)
            def _():
                _fire_meta(b + 1, 1 - q)

            pltpu.async_copy(ncat.at[gvq], rowsq, sem_g).wait()
            pltpu.async_copy(rowsq, tacc.at[jvq], sem_s, add=True)

            @pl.when(cid == 0)
            def _():
                pltpu.async_copy(nv, sacc.at[sivq], sem_s, add=True)
                pltpu.async_copy(onesb, sacc.at[civq], sem_s, add=True)

        return 0

    lax.fori_loop(0, NB // 2, _group, 0)
    for q in (0, 1):
        _wait_scatter(q)
    plsc.subcore_barrier()

    def _tcopy(i, _):
        base = sid * RPT + i * CB
        pltpu.sync_copy(tacc.at[pl.ds(base, CB)], cbuf)
        pltpu.sync_copy(cbuf, t_out.at[pl.ds(cid * TWO_N + base, CB)])
        return 0

    lax.fori_loop(0, RPT // CB, _tcopy, 0)

    @pl.when(cid == 0)
    def _():
        def _scopy(i, _):
            base = sid * SPT + i * SCB
            pltpu.sync_copy(sacc.at[pl.ds(base, SCB)], sbuf)
            pltpu.sync_copy(sbuf, s_out.at[pl.ds(base, SCB)])
            return 0

        lax.fori_loop(0, SPT // SCB, _scopy, 0)


_sc_aggregate = pl.kernel(
    _sc_body,
    out_type=(
        jax.ShapeDtypeStruct((2 * TWO_N, DH), jnp.float32),
        jax.ShapeDtypeStruct((TWO_N * SW,), jnp.float32),
    ),
    mesh=plsc.VectorSubcoreMesh(core_axis_name="c", subcore_axis_name="s"),
    compiler_params=pltpu.CompilerParams(use_tc_tiling_on_sc=False),
    scratch_types=[
        pltpu.VMEM((B,), jnp.int32),
        pltpu.VMEM((B,), jnp.int32),
        pltpu.VMEM((B,), jnp.int32),
        pltpu.VMEM((B,), jnp.int32),
        pltpu.VMEM((B,), jnp.int32),
        pltpu.VMEM((B,), jnp.int32),
        pltpu.VMEM((B,), jnp.int32),
        pltpu.VMEM((B,), jnp.int32),
        pltpu.VMEM((B,), jnp.float32),
        pltpu.VMEM((B,), jnp.float32),
        pltpu.VMEM((B,), jnp.int32),
        pltpu.VMEM((B,), jnp.int32),
        pltpu.VMEM((B,), jnp.int32),
        pltpu.VMEM((B,), jnp.int32),
        pltpu.VMEM((B,), jnp.int32),
        pltpu.VMEM((B,), jnp.int32),
        pltpu.VMEM((B,), jnp.int32),
        pltpu.VMEM((B,), jnp.int32),
        pltpu.VMEM((B, DH), jnp.float32),
        pltpu.VMEM((B, DH), jnp.float32),
        pltpu.VMEM((B,), jnp.float32),
        pltpu.VMEM((CB, DH), jnp.float32),
        pltpu.VMEM((SCB,), jnp.float32),
        pltpu.VMEM_SHARED((TWO_N, DH), jnp.float32),
        pltpu.VMEM_SHARED((TWO_N * SW,), jnp.float32),
        pltpu.SemaphoreType.DMA,
        pltpu.SemaphoreType.DMA,
        pltpu.SemaphoreType.DMA,
        pltpu.SemaphoreType.DMA,
    ],
)


def _tc_body(nin, t, s, rf, loop, wo, bo, wi, bi, ws, bs, wr, br, g, bb,
             nout, rout):
    r = rf[...]
    ao = jnp.concatenate([t[0:N], t[TWO_N:TWO_N + N]], axis=1)
    ai = jnp.concatenate([t[N:TWO_N], t[TWO_N + N:2 * TWO_N]], axis=1)
    so = s[0:N, 0:R]
    si = s[N:TWO_N, 0:R]
    co = s[0:N, R:R + 1]
    ci = s[N:TWO_N, R:R + 1]
    mo = ao - jnp.dot(so, r, preferred_element_type=jnp.float32)
    mi = ai - jnp.dot(si, r, preferred_element_type=jnp.float32)
    comp = (jnp.dot(mo, wo[...], preferred_element_type=jnp.float32)
            + jnp.dot(mi, wi[...], preferred_element_type=jnp.float32)
            + co * bo[...] + ci * bi[...])
    h = jnp.dot(nin[...] - loop[...], ws[...],
                preferred_element_type=jnp.float32) + bs[...] + comp
    h = h * (1.0 / 3.0)
    mean = jnp.mean(h, axis=0, keepdims=True)
    var = jnp.mean((h - mean) ** 2, axis=0, keepdims=True)
    y = (h - mean) * lax.rsqrt(var + EPS_) * g[...] + bb[...]
    nout[...] = jnp.tanh(y)
    rout[...] = jnp.dot(r, wr[...], preferred_element_type=jnp.float32) + br[...]


_tc_finish = pl.pallas_call(
    _tc_body,
    out_shape=(
        jax.ShapeDtypeStruct((N, D_OUT), jnp.float32),
        jax.ShapeDtypeStruct((R, D_OUT), jnp.float32),
    ),
)


def kernel(n_in_feats, r_feats, edge_src, edge_dst, etype, norm,
           out_edges_mask, in_edges_mask,
           W_O, b_O, W_I, b_I, W_S, b_S, W_R, b_R,
           loop_rel, bn_gamma, bn_beta):
    src = edge_src.astype(jnp.int32)
    dst = edge_dst.astype(jnp.int32)
    et = etype.astype(jnp.int32)
    dirv = out_edges_mask.astype(jnp.int32)
    nrm = norm.reshape(E)
    ncat = jnp.concatenate([n_in_feats[:, :DH], n_in_feats[:, DH:]], axis=0)
    t, s = _sc_aggregate(ncat, src, dst, dirv, et, nrm)
    s2 = s.reshape(TWO_N, SW)
    n_out, r_out = _tc_finish(n_in_feats, t, s2, r_feats, loop_rel,
                              W_O, b_O, W_I, b_I, W_S, b_S, W_R, b_R,
                              bn_gamma, bn_beta)
    return n_out, r_out


# speedup vs baseline: 11.0586x; 1.2362x over previous
import jax
import jax.numpy as jnp
from jax import lax
from jax.experimental import pallas as pl
from jax.experimental.pallas import tpu as pltpu
from jax.experimental.pallas import tpu_sc as plsc

N = 10000
E = 320000
D_IN = 128
D_OUT = 128
R = 16
EPS_ = 1e-5
DH = 64
TWO_N = 2 * N
SW = 20
NTILES = 16
EPT = E // NTILES
B = 80
NB = EPT // B
RPT = TWO_N // NTILES
CB = 125
SPT = (TWO_N * SW) // NTILES
SCB = 1000


def _sc_body(ncat, src_a, dst_a, dir_a, et_a, nrm_a, t_out, s_out,
             srcv0, srcv1, dstv0, dstv1, dirv0, dirv1, etv0, etv1,
             nrmv0, nrmv1, gv0, gv1, jv0, jv1, siv0, siv1, civ0, civ1,
             rows0, rows1, nsrc0, nsrc1, onesb, cbuf, sbuf, tacc, sacc,
             sem_m, sem_g0, sem_g1, sem_s0, sem_s1):
    cid = lax.axis_index("c")
    sid = lax.axis_index("s")
    zf = jnp.zeros((16,), jnp.float32)
    ones16 = jnp.ones((16,), jnp.float32)

    for k in range(B // 16):
        onesb[pl.ds(16 * k, 16)] = ones16

    def _zrow(r_, _):
        for l in range(DH // 16):
            cbuf[r_, pl.ds(l * 16, 16)] = zf
        return 0

    lax.fori_loop(0, CB, _zrow, 0)

    def _zsbuf(i, _):
        sbuf[pl.ds(i * 16, 16)] = zf
        return 0

    lax.fori_loop(0, SCB // 16, _zsbuf, 0)

    def _zcopy(i, _):
        base = sid * RPT + i * CB
        pltpu.sync_copy(cbuf, tacc.at[pl.ds(base, CB)])
        return 0

    lax.fori_loop(0, RPT // CB, _zcopy, 0)

    def _zscopy(i, _):
        pltpu.sync_copy(sbuf, sacc.at[pl.ds(sid * SPT + i * SCB, SCB)])
        return 0

    lax.fori_loop(0, SPT // SCB, _zscopy, 0)
    plsc.subcore_barrier()

    bufsets = (
        (srcv0, dstv0, dirv0, etv0, nrmv0, gv0, jv0, siv0, civ0,
         rows0, nsrc0, sem_g0, sem_s0),
        (srcv1, dstv1, dirv1, etv1, nrmv1, gv1, jv1, siv1, civ1,
         rows1, nsrc1, sem_g1, sem_s1),
    )

    def _fire_meta(bn, q):
        sv, dv, rv, ev, nv = bufsets[q][0:5]
        o = sid * EPT + bn * B
        pltpu.async_copy(src_a.at[pl.ds(o, B)], sv, sem_m)
        pltpu.async_copy(dst_a.at[pl.ds(o, B)], dv, sem_m)
        pltpu.async_copy(dir_a.at[pl.ds(o, B)], rv, sem_m)
        pltpu.async_copy(et_a.at[pl.ds(o, B)], ev, sem_m)
        pltpu.async_copy(nrm_a.at[pl.ds(o, B)], nv, sem_m)

    def _wait_meta(q):
        sv, dv, rv, ev, nv = bufsets[q][0:5]
        pltpu.make_async_copy(src_a.at[pl.ds(0, B)], sv, sem_m).wait()
        pltpu.make_async_copy(dst_a.at[pl.ds(0, B)], dv, sem_m).wait()
        pltpu.make_async_copy(dir_a.at[pl.ds(0, B)], rv, sem_m).wait()
        pltpu.make_async_copy(et_a.at[pl.ds(0, B)], ev, sem_m).wait()
        pltpu.make_async_copy(nrm_a.at[pl.ds(0, B)], nv, sem_m).wait()

    def _wait_scatter(q):
        jvq, sivq, civq, rowsq, nsrcq, sem_s = bufsets[q][6:11] + bufsets[q][12:13]
        pltpu.make_async_copy(rowsq, tacc.at[jvq], sem_s).wait()

        @pl.when(cid == 0)
        def _():
            pltpu.make_async_copy(nsrcq, sacc.at[sivq], sem_s).wait()
            pltpu.make_async_copy(onesb, sacc.at[civq], sem_s).wait()

    def _fire_scatter(q):
        jvq, sivq, civq, rowsq, nsrcq, sem_s = bufsets[q][6:11] + bufsets[q][12:13]
        pltpu.async_copy(rowsq, tacc.at[jvq], sem_s, add=True)

        @pl.when(cid == 0)
        def _():
            pltpu.async_copy(nsrcq, sacc.at[sivq], sem_s, add=True)
            pltpu.async_copy(onesb, sacc.at[civq], sem_s, add=True)

    def _wait_gather(q):
        gvq, _, _, _, rowsq, _, sem_g = bufsets[q][5:12]
        pltpu.make_async_copy(ncat.at[gvq], rowsq, sem_g).wait()

    _fire_meta(0, 0)

    def _group(g, _):
        for q in (0, 1):
            b = g * 2 + q
            (sv, dv, rv, ev, nv, gvq, jvq, sivq, civq,
             rowsq, nsrcq, sem_g, sem_s) = bufsets[q]
            _wait_meta(q)

            @pl.when(g >= 1)
            def _():
                _wait_scatter(q)

            for k in range(B // 16):
                ds16 = pl.ds(16 * k, 16)
                j16 = dv[ds16] + (1 - rv[ds16]) * N
                gvq[ds16] = sv[ds16] + cid * N
                jvq[ds16] = j16
                sivq[ds16] = j16 * SW + ev[ds16]
                civq[ds16] = j16 * SW + R
                nsrcq[ds16] = nv[ds16]

            if q == 0:
                _fire_meta(b + 1, 1)
                pltpu.async_copy(ncat.at[gvq], rowsq, sem_g)

                @pl.when(g >= 1)
                def _():
                    _wait_gather(1)
                    _fire_scatter(1)
            else:
                @pl.when(g < NB // 2 - 1)
                def _():
                    _fire_meta(b + 1, 0)

                pltpu.async_copy(ncat.at[gvq], rowsq, sem_g)
                _wait_gather(0)
                _fire_scatter(0)

        return 0

    lax.fori_loop(0, NB // 2, _group, 0)
    _wait_gather(1)
    _fire_scatter(1)
    _wait_scatter(0)
    _wait_scatter(1)
    plsc.subcore_barrier()

    def _tcopy(i, _):
        base = sid * RPT + i * CB
        pltpu.sync_copy(tacc.at[pl.ds(base, CB)], cbuf)
        pltpu.sync_copy(cbuf, t_out.at[pl.ds(cid * TWO_N + base, CB)])
        return 0

    lax.fori_loop(0, RPT // CB, _tcopy, 0)

    @pl.when(cid == 0)
    def _():
        def _scopy(i, _):
            base = sid * SPT + i * SCB
            pltpu.sync_copy(sacc.at[pl.ds(base, SCB)], sbuf)
            pltpu.sync_copy(sbuf, s_out.at[pl.ds(base, SCB)])
            return 0

        lax.fori_loop(0, SPT // SCB, _scopy, 0)


_sc_aggregate = pl.kernel(
    _sc_body,
    out_type=(
        jax.ShapeDtypeStruct((2 * TWO_N, DH), jnp.float32),
        jax.ShapeDtypeStruct((TWO_N * SW,), jnp.float32),
    ),
    mesh=plsc.VectorSubcoreMesh(core_axis_name="c", subcore_axis_name="s"),
    compiler_params=pltpu.CompilerParams(use_tc_tiling_on_sc=False),
    scratch_types=[
        pltpu.VMEM((B,), jnp.int32),
        pltpu.VMEM((B,), jnp.int32),
        pltpu.VMEM((B,), jnp.int32),
        pltpu.VMEM((B,), jnp.int32),
        pltpu.VMEM((B,), jnp.int32),
        pltpu.VMEM((B,), jnp.int32),
        pltpu.VMEM((B,), jnp.int32),
        pltpu.VMEM((B,), jnp.int32),
        pltpu.VMEM((B,), jnp.float32),
        pltpu.VMEM((B,), jnp.float32),
        pltpu.VMEM((B,), jnp.int32),
        pltpu.VMEM((B,), jnp.int32),
        pltpu.VMEM((B,), jnp.int32),
        pltpu.VMEM((B,), jnp.int32),
        pltpu.VMEM((B,), jnp.int32),
        pltpu.VMEM((B,), jnp.int32),
        pltpu.VMEM((B,), jnp.int32),
        pltpu.VMEM((B,), jnp.int32),
        pltpu.VMEM((B, DH), jnp.float32),
        pltpu.VMEM((B, DH), jnp.float32),
        pltpu.VMEM((B,), jnp.float32),
        pltpu.VMEM((B,), jnp.float32),
        pltpu.VMEM((B,), jnp.float32),
        pltpu.VMEM((CB, DH), jnp.float32),
        pltpu.VMEM((SCB,), jnp.float32),
        pltpu.VMEM_SHARED((TWO_N, DH), jnp.float32),
        pltpu.VMEM_SHARED((TWO_N * SW,), jnp.float32),
        pltpu.SemaphoreType.DMA,
        pltpu.SemaphoreType.DMA,
        pltpu.SemaphoreType.DMA,
        pltpu.SemaphoreType.DMA,
        pltpu.SemaphoreType.DMA,
    ],
)


def _tc_body(nin, t, s, rf, loop, wo, bo, wi, bi, ws, bs, wr, br, g, bb,
             nout, rout):
    r = rf[...]
    ao = jnp.concatenate([t[0:N], t[TWO_N:TWO_N + N]], axis=1)
    ai = jnp.concatenate([t[N:TWO_N], t[TWO_N + N:2 * TWO_N]], axis=1)
    so = s[0:N, 0:R]
    si = s[N:TWO_N, 0:R]
    co = s[0:N, R:R + 1]
    ci = s[N:TWO_N, R:R + 1]
    mo = ao - jnp.dot(so, r, preferred_element_type=jnp.float32)
    mi = ai - jnp.dot(si, r, preferred_element_type=jnp.float32)
    comp = (jnp.dot(mo, wo[...], preferred_element_type=jnp.float32)
            + jnp.dot(mi, wi[...], preferred_element_type=jnp.float32)
            + co * bo[...] + ci * bi[...])
    h = jnp.dot(nin[...] - loop[...], ws[...],
                preferred_element_type=jnp.float32) + bs[...] + comp
    h = h * (1.0 / 3.0)
    mean = jnp.mean(h, axis=0, keepdims=True)
    var = jnp.mean((h - mean) ** 2, axis=0, keepdims=True)
    y = (h - mean) * lax.rsqrt(var + EPS_) * g[...] + bb[...]
    nout[...] = jnp.tanh(y)
    rout[...] = jnp.dot(r, wr[...], preferred_element_type=jnp.float32) + br[...]


_tc_finish = pl.pallas_call(
    _tc_body,
    out_shape=(
        jax.ShapeDtypeStruct((N, D_OUT), jnp.float32),
        jax.ShapeDtypeStruct((R, D_OUT), jnp.float32),
    ),
)


def kernel(n_in_feats, r_feats, edge_src, edge_dst, etype, norm,
           out_edges_mask, in_edges_mask,
           W_O, b_O, W_I, b_I, W_S, b_S, W_R, b_R,
           loop_rel, bn_gamma, bn_beta):
    src = edge_src.astype(jnp.int32)
    dst = edge_dst.astype(jnp.int32)
    et = etype.astype(jnp.int32)
    dirv = out_edges_mask.astype(jnp.int32)
    nrm = norm.reshape(E)
    ncat = jnp.concatenate([n_in_feats[:, :DH], n_in_feats[:, DH:]], axis=0)
    t, s = _sc_aggregate(ncat, src, dst, dirv, et, nrm)
    s2 = s.reshape(TWO_N, SW)
    n_out, r_out = _tc_finish(n_in_feats, t, s2, r_feats, loop_rel,
                              W_O, b_O, W_I, b_I, W_S, b_S, W_R, b_R,
                              bn_gamma, bn_beta)
    return n_out, r_out
